# Initial kernel scaffold; baseline (speedup 1.0000x reference)
import jax
import jax.numpy as jnp
from jax import lax
from jax.experimental import pallas as pl
from jax.experimental.pallas import tpu as pltpu
from jax.experimental.pallas import tpu_sc as plsc

_N = 16
_M = 64
_B = 1048576

_NW = 32
_BPW = _B // _NW
_CH = 2048
_NCHUNK = _BPW // _CH


def _logsm_body(mt_ref, out_ref):
    x = mt_ref[...]
    mx = jnp.max(x, axis=0, keepdims=True)
    s = x - mx
    lse = jnp.log(jnp.sum(jnp.exp(s), axis=0, keepdims=True))
    out_ref[...] = s - lse


def _make_table(matrix):
    return pl.pallas_call(
        _logsm_body,
        out_shape=jax.ShapeDtypeStruct((_M, _N), jnp.float32),
    )(matrix.T)


def _gather_body(table_hbm, xt_hbm, out_hbm, idx_v, rows_v, sem):
    wid = lax.axis_index("s") * 2 + lax.axis_index("c")
    base = wid * _BPW

    def step(i, carry):
        off = base + i * _CH
        pltpu.sync_copy(xt_hbm.at[pl.ds(off, _CH)], idx_v)
        pltpu.async_copy(table_hbm.at[idx_v], rows_v, sem).wait()
        pltpu.sync_copy(rows_v, out_hbm.at[pl.ds(off, _CH)])
        return carry

    lax.fori_loop(0, _NCHUNK, step, 0)


def kernel(matrix, x_t):
    table = _make_table(matrix)
    f = pl.kernel(
        _gather_body,
        out_type=jax.ShapeDtypeStruct((_B, _N), jnp.float32),
        mesh=plsc.VectorSubcoreMesh(core_axis_name="c", subcore_axis_name="s"),
        scratch_types=[
            pltpu.VMEM((_CH,), jnp.int32),
            pltpu.VMEM((_CH, _N), jnp.float32),
            pltpu.SemaphoreType.DMA,
        ],
    )
    return f(table, x_t)


# speedup vs baseline: 3.5974x; 3.5974x over previous
import jax
import jax.numpy as jnp
from jax import lax
from jax.experimental import pallas as pl
from jax.experimental.pallas import tpu as pltpu
from jax.experimental.pallas import tpu_sc as plsc

_N = 16
_M = 64
_B = 1048576

_NW = 32
_BPW = _B // _NW
_CH = 2048
_NCHUNK = _BPW // _CH


def _logsm_body(mt_ref, out_ref):
    x = mt_ref[...]
    mx = jnp.max(x, axis=0, keepdims=True)
    s = x - mx
    lse = jnp.log(jnp.sum(jnp.exp(s), axis=0, keepdims=True))
    out_ref[...] = s - lse


def _make_table(matrix):
    return pl.pallas_call(
        _logsm_body,
        out_shape=jax.ShapeDtypeStruct((_M, _N), jnp.float32),
    )(matrix.T)


def _gather_body(table_hbm, xt_hbm, out_hbm, idx_v, rows_v, sem):
    wid = lax.axis_index("s") * 2 + lax.axis_index("c")
    base = wid * _BPW

    def step(i, carry):
        off = base + i * _CH
        pltpu.sync_copy(xt_hbm.at[pl.ds(off, _CH)], idx_v)
        pltpu.async_copy(table_hbm.at[idx_v], rows_v, sem).wait()
        pltpu.sync_copy(rows_v, out_hbm.at[pl.ds(off, _CH)])
        return carry

    lax.fori_loop(0, _NCHUNK, step, 0)


def kernel(matrix, x_t):
    table = _make_table(matrix)
    f = pl.kernel(
        _gather_body,
        out_type=jax.ShapeDtypeStruct((_B, _N), jnp.float32),
        mesh=plsc.VectorSubcoreMesh(core_axis_name="c", subcore_axis_name="s"),
        scratch_types=[
            pltpu.VMEM((_CH,), jnp.int32),
            pltpu.VMEM((_CH, _N), jnp.float32),
            pltpu.SemaphoreType.DMA,
        ],
        compiler_params=pltpu.CompilerParams(use_tc_tiling_on_sc=False),
    )
    return f(table, x_t)


# speedup vs baseline: 7.5871x; 2.1090x over previous
import jax
import jax.numpy as jnp
from jax import lax
from jax.experimental import pallas as pl
from jax.experimental.pallas import tpu as pltpu
from jax.experimental.pallas import tpu_sc as plsc

_N = 16
_M = 64
_B = 1048576

_NW = 32
_BPW = _B // _NW
_CH = 2048
_NCHUNK = _BPW // _CH


def _logsm_body(mt_ref, out_ref):
    x = mt_ref[...]
    mx = jnp.max(x, axis=0, keepdims=True)
    s = x - mx
    lse = jnp.log(jnp.sum(jnp.exp(s), axis=0, keepdims=True))
    out_ref[...] = s - lse


def _make_table(matrix):
    return pl.pallas_call(
        _logsm_body,
        out_shape=jax.ShapeDtypeStruct((_M, _N), jnp.float32),
    )(matrix.T)


def _gather_body(table_hbm, xt_hbm, out_hbm, table_sp, idx_v, rows_v, sem):
    sid = lax.axis_index("s")
    wid = sid * 2 + lax.axis_index("c")
    base = wid * _BPW

    @pl.when(sid == 0)
    def _():
        pltpu.sync_copy(table_hbm, table_sp)

    plsc.subcore_barrier()

    def step(i, carry):
        off = base + i * _CH
        pltpu.sync_copy(xt_hbm.at[pl.ds(off, _CH)], idx_v)
        pltpu.async_copy(table_sp.at[idx_v], rows_v, sem).wait()
        pltpu.sync_copy(rows_v, out_hbm.at[pl.ds(off, _CH)])
        return carry

    lax.fori_loop(0, _NCHUNK, step, 0)


def kernel(matrix, x_t):
    table = _make_table(matrix)
    f = pl.kernel(
        _gather_body,
        out_type=jax.ShapeDtypeStruct((_B, _N), jnp.float32),
        mesh=plsc.VectorSubcoreMesh(core_axis_name="c", subcore_axis_name="s"),
        scratch_types=[
            pltpu.VMEM_SHARED((_M, _N), jnp.float32),
            pltpu.VMEM((_CH,), jnp.int32),
            pltpu.VMEM((_CH, _N), jnp.float32),
            pltpu.SemaphoreType.DMA,
        ],
        compiler_params=pltpu.CompilerParams(use_tc_tiling_on_sc=False),
    )
    return f(table, x_t)


# speedup vs baseline: 7.8990x; 1.0411x over previous
import jax
import jax.numpy as jnp
from jax import lax
from jax.experimental import pallas as pl
from jax.experimental.pallas import tpu as pltpu
from jax.experimental.pallas import tpu_sc as plsc

_N = 16
_M = 64
_B = 1048576

_NW = 32
_BPW = _B // _NW
_CH = 2048
_NCHUNK = _BPW // _CH


def _logsm_body(mt_ref, out_ref):
    x = mt_ref[...]
    mx = jnp.max(x, axis=0, keepdims=True)
    s = x - mx
    lse = jnp.log(jnp.sum(jnp.exp(s), axis=0, keepdims=True))
    out_ref[...] = s - lse


def _make_table(matrix):
    return pl.pallas_call(
        _logsm_body,
        out_shape=jax.ShapeDtypeStruct((_M, _N), jnp.float32),
    )(matrix.T)


_NBUF = 3


def _gather_body(table_hbm, xt_hbm, out_hbm, table_sp, idx_v, rows_v, *sems):
    si = sems[0:_NBUF]
    sg = sems[_NBUF:2 * _NBUF]
    sw = sems[2 * _NBUF:3 * _NBUF]
    sid = lax.axis_index("s")
    wid = sid * 2 + lax.axis_index("c")
    base = wid * _BPW

    @pl.when(sid == 0)
    def _():
        pltpu.sync_copy(table_hbm, table_sp)

    plsc.subcore_barrier()

    def start_idx(b, i):
        return pltpu.async_copy(
            xt_hbm.at[pl.ds(base + i * _CH, _CH)], idx_v.at[b], si[b])

    def start_gather(b):
        return pltpu.async_copy(table_sp.at[idx_v.at[b]], rows_v.at[b], sg[b])

    def start_write(b, i):
        return pltpu.async_copy(
            rows_v.at[b], out_hbm.at[pl.ds(base + i * _CH, _CH)], sw[b])

    h_idx = [None] * _NBUF
    h_g = [None] * _NBUF
    h_w = [None] * _NBUF
    for i in range(_NBUF):
        h_idx[i] = start_idx(i, i)
    for i in range(_NCHUNK):
        b = i % _NBUF
        h_idx[b].wait()
        if i >= _NBUF:
            h_w[b].wait()
        h_g[b] = start_gather(b)
        if i >= 1:
            pb = (i - 1) % _NBUF
            h_g[pb].wait()
            h_w[pb] = start_write(pb, i - 1)
            if (i - 1) + _NBUF < _NCHUNK:
                h_idx[pb] = start_idx(pb, i - 1 + _NBUF)
    lb = (_NCHUNK - 1) % _NBUF
    h_g[lb].wait()
    h_w[lb] = start_write(lb, _NCHUNK - 1)
    for b in range(_NBUF):
        h_w[b].wait()


def kernel(matrix, x_t):
    table = _make_table(matrix)
    f = pl.kernel(
        _gather_body,
        out_type=jax.ShapeDtypeStruct((_B, _N), jnp.float32),
        mesh=plsc.VectorSubcoreMesh(core_axis_name="c", subcore_axis_name="s"),
        scratch_types=[
            pltpu.VMEM_SHARED((_M, _N), jnp.float32),
            pltpu.VMEM((_NBUF, _CH), jnp.int32),
            pltpu.VMEM((_NBUF, _CH, _N), jnp.float32),
        ] + [pltpu.SemaphoreType.DMA] * (3 * _NBUF),
        compiler_params=pltpu.CompilerParams(use_tc_tiling_on_sc=False),
    )
    return f(table, x_t)
